# baseline (device time: 32682 ns/iter reference)
import jax
import jax.numpy as jnp
from jax import lax
from jax.experimental import pallas as pl
from jax.experimental.pallas import tpu as pltpu

N_DEV = 32
SQ = 256
D = 1024
H = 8
DH = 128
C = SQ // N_DEV
SCALE = 0.08838834764831843


def _allreduce(partial_bf):
    m, n = partial_bf.shape
    bf = jnp.bfloat16
    f32 = jnp.float32

    def body(in_ref, out_ref, rs_hbuf, rs_vbuf, gbuf, copy_sem,
             rs_send, rs_recv, ag_send, ag_recv):
        me = lax.axis_index("i")

        barrier_sem = pltpu.get_barrier_semaphore()
        pl.semaphore_signal(barrier_sem, inc=1)
        pl.semaphore_wait(barrier_sem, 1)

        rs = []
        for j in range(N_DEV - 1):
            p = (me + 1 + j) & (N_DEV - 1)
            rdma = pltpu.make_async_remote_copy(
                src_ref=in_ref.at[pl.ds(pl.multiple_of(p * C, 8), C), :],
                dst_ref=rs_hbuf.at[j],
                send_sem=rs_send.at[j],
                recv_sem=rs_recv.at[j],
                device_id=(p,),
                device_id_type=pl.DeviceIdType.MESH,
            )
            rdma.start()
            rs.append(rdma)
        for rdma in rs:
            rdma.wait()

        local = pltpu.make_async_copy(rs_hbuf, rs_vbuf, copy_sem)
        local.start()
        local.wait()
        my_row = pl.multiple_of(me * C, 8)
        acc = in_ref[pl.ds(my_row, C), :].astype(f32)
        for j in range(N_DEV - 1):
            acc = acc + rs_vbuf[j].astype(f32)
        gbuf[pl.ds(my_row, C), :] = acc.astype(bf)

        ag = []
        for j in range(N_DEV - 1):
            p = (me + 1 + j) & (N_DEV - 1)
            rdma = pltpu.make_async_remote_copy(
                src_ref=gbuf.at[pl.ds(my_row, C), :],
                dst_ref=gbuf.at[pl.ds(my_row, C), :],
                send_sem=ag_send.at[j],
                recv_sem=ag_recv.at[j],
                device_id=(p,),
                device_id_type=pl.DeviceIdType.MESH,
            )
            rdma.start()
            ag.append(rdma)
        for rdma in ag:
            rdma.wait()
        out_ref[...] = gbuf[...].astype(f32)

    out, _ = pl.pallas_call(
        body,
        out_shape=(
            jax.ShapeDtypeStruct((m, n), f32),
            jax.ShapeDtypeStruct((N_DEV - 1, C, n), bf),
        ),
        in_specs=[pl.BlockSpec(memory_space=pltpu.VMEM)],
        out_specs=(
            pl.BlockSpec(memory_space=pltpu.VMEM),
            pl.BlockSpec(memory_space=pl.ANY),
        ),
        scratch_shapes=[
            pltpu.VMEM((N_DEV - 1, C, n), bf),
            pltpu.VMEM((m, n), bf),
            pltpu.SemaphoreType.DMA,
            pltpu.SemaphoreType.DMA((N_DEV - 1,)),
            pltpu.SemaphoreType.DMA((N_DEV - 1,)),
            pltpu.SemaphoreType.DMA((N_DEV - 1,)),
            pltpu.SemaphoreType.DMA((N_DEV - 1,)),
        ],
        compiler_params=pltpu.CompilerParams(collective_id=0),
    )(partial_bf)
    return out


def kernel(x, Wq, Wo, Wk, Wv):
    bf = jnp.bfloat16
    f32 = jnp.float32
    xb = x.reshape(SQ, D).astype(bf)
    q = jnp.dot(xb, Wq.astype(bf), preferred_element_type=bf)
    k = jnp.dot(xb, Wk.astype(bf), preferred_element_type=bf)
    v = jnp.dot(xb, Wv.astype(bf), preferred_element_type=bf)
    q = q.reshape(SQ, H, DH)
    k = k.reshape(SQ, H, DH)
    v = v.reshape(SQ, H, DH)
    s = jnp.einsum("ihd,jhd->hij", q, k, preferred_element_type=f32) * SCALE
    p = jax.nn.softmax(s, axis=-1).astype(bf)
    o = jnp.einsum("hij,jhd->ihd", p, v, preferred_element_type=bf)
    partial_bf = jnp.dot(
        o.reshape(SQ, H * DH), Wo.astype(bf), preferred_element_type=f32
    ).astype(bf)
    out = _allreduce(partial_bf)
    return out.reshape(1, SQ, D)
